# initial kernel scaffold (unmeasured)
import jax
import jax.numpy as jnp
from jax import lax
from jax.experimental import pallas as pl
from jax.experimental.pallas import tpu as pltpu


def kernel(
    x,
):
    def body(*refs):
        pass

    out_shape = jax.ShapeDtypeStruct(..., jnp.float32)
    return pl.pallas_call(body, out_shape=out_shape)(...)


# baseline (device time: 68321 ns/iter reference)
import jax
import jax.numpy as jnp
from jax import lax
from jax.experimental import pallas as pl
from jax.experimental.pallas import tpu as pltpu

N_DEV = 32
LOG_M = 8
N_CROSS = 15


def kernel(x):
    m, n = x.shape
    assert m == 1 << LOG_M

    def body(x_ref, o_ref, cur_ref, recv_ref, send_sems, recv_sems):
        pos = lax.axis_index("i")
        lidx = lax.broadcasted_iota(jnp.int32, (m, n), 0)

        def local_stage(xv, j, asc):
            up = jnp.roll(xv, -j, axis=0)
            down = jnp.roll(xv, j, axis=0)
            lower = (lidx & j) == 0
            partner = jnp.where(lower, up, down)
            return jnp.where(
                asc == lower,
                jnp.minimum(xv, partner),
                jnp.maximum(xv, partner),
            )

        xv = x_ref[:, :]

        for lk in range(1, LOG_M):
            k = 1 << lk
            asc = (lidx & k) == 0
            for lj in range(lk - 1, -1, -1):
                xv = local_stage(xv, 1 << lj, asc)

        asc_s = (pos & 1) == 0
        for lj in range(LOG_M - 1, -1, -1):
            xv = local_stage(xv, 1 << lj, asc_s)

        s = 0
        for lk in range(LOG_M + 1, 14):
            kb = 1 << (lk - LOG_M)
            asc_s = (pos & kb) == 0
            for lj in range(lk - 1, LOG_M - 1, -1):
                jb = 1 << (lj - LOG_M)
                partner = pos ^ jb
                cur_ref[:, :] = xv
                rdma = pltpu.make_async_remote_copy(
                    src_ref=cur_ref,
                    dst_ref=recv_ref.at[s],
                    send_sem=send_sems.at[s],
                    recv_sem=recv_sems.at[s],
                    device_id=(partner,),
                    device_id_type=pl.DeviceIdType.MESH,
                )
                rdma.start()
                rdma.wait()
                other = recv_ref[s]
                lower_s = (pos & jb) == 0
                choose_min = asc_s == lower_s
                xv = jnp.where(
                    choose_min,
                    jnp.minimum(xv, other),
                    jnp.maximum(xv, other),
                )
                s += 1
            for lj in range(LOG_M - 1, -1, -1):
                xv = local_stage(xv, 1 << lj, asc_s)

        o_ref[:, :] = xv

    return pl.pallas_call(
        body,
        out_shape=jax.ShapeDtypeStruct((m, n), x.dtype),
        in_specs=[pl.BlockSpec(memory_space=pltpu.VMEM)],
        out_specs=pl.BlockSpec(memory_space=pltpu.VMEM),
        scratch_shapes=[
            pltpu.VMEM((m, n), x.dtype),
            pltpu.VMEM((N_CROSS, m, n), x.dtype),
            pltpu.SemaphoreType.DMA((N_CROSS,)),
            pltpu.SemaphoreType.DMA((N_CROSS,)),
        ],
    )(x)


# device time: 34051 ns/iter; 2.0064x vs baseline; 2.0064x over previous
import jax
import jax.numpy as jnp
from jax import lax
from jax.experimental import pallas as pl
from jax.experimental.pallas import tpu as pltpu

N_DEV = 32
C_PER = 4
LOG_TOT = 13


def kernel(x):
    m, n = x.shape
    assert m == 256 and n == 128

    wn, wm = n, m

    def body(x_ref, o_ref, xt_ref, w_ref, ot_ref, sf_send, sf_recv, sb_send, sb_recv):
        pos = lax.axis_index("i")

        def drain(send_sem, recv_sem, count, which):
            dummy = pltpu.make_async_remote_copy(
                src_ref=xt_ref.at[pl.ds(0, C_PER)],
                dst_ref=xt_ref.at[pl.ds(0, C_PER)],
                send_sem=send_sem,
                recv_sem=recv_sem,
                device_id=(pos,),
                device_id_type=pl.DeviceIdType.MESH,
            )
            for _ in range(count):
                if which == "recv":
                    dummy.wait_recv()
                else:
                    dummy.wait_send()

        xt_ref[:, :] = x_ref[:, :].T

        for q in range(N_DEV):
            copy = pltpu.make_async_remote_copy(
                src_ref=xt_ref.at[pl.ds(C_PER * q, C_PER)],
                dst_ref=w_ref.at[pl.ds(C_PER * pos, C_PER)],
                send_sem=sf_send,
                recv_sem=sf_recv,
                device_id=(q,),
                device_id_type=pl.DeviceIdType.MESH,
            )

            copy.start()

        drain(sf_send, sf_recv, N_DEV, "recv")

        row = lax.broadcasted_iota(jnp.int32, (wn, wm), 0)
        lane = lax.broadcasted_iota(jnp.int32, (wn, wm), 1)
        fidx = (row // C_PER) * wm + lane

        w = w_ref[:, :]
        for lk in range(1, LOG_TOT + 1):
            k = 1 << lk
            asc = (fidx & k) == 0
            for lj in range(lk - 1, -1, -1):
                j = 1 << lj
                if j >= wm:
                    d = C_PER * (j // wm)
                    up = jnp.roll(w, -d, axis=0)
                    down = jnp.roll(w, d, axis=0)
                else:
                    la = jnp.roll(w, -j, axis=1)
                    up = jnp.where(lane < wm - j, la, jnp.roll(la, -C_PER, axis=0))
                    lb = jnp.roll(w, j, axis=1)
                    down = jnp.where(lane >= j, lb, jnp.roll(lb, C_PER, axis=0))
                lower = (fidx & j) == 0
                partner = jnp.where(lower, up, down)
                w = jnp.where(
                    asc == lower,
                    jnp.minimum(w, partner),
                    jnp.maximum(w, partner),
                )
        w_ref[:, :] = w

        for p in range(N_DEV):
            copy = pltpu.make_async_remote_copy(
                src_ref=w_ref.at[pl.ds(C_PER * p, C_PER)],
                dst_ref=ot_ref.at[pl.ds(C_PER * pos, C_PER)],
                send_sem=sb_send,
                recv_sem=sb_recv,
                device_id=(p,),
                device_id_type=pl.DeviceIdType.MESH,
            )

            copy.start()

        drain(sb_send, sb_recv, N_DEV, "recv")
        o_ref[:, :] = ot_ref[:, :].T

        drain(sf_send, sf_recv, N_DEV, "send")
        drain(sb_send, sb_recv, N_DEV, "send")

    return pl.pallas_call(
        body,
        out_shape=jax.ShapeDtypeStruct((m, n), x.dtype),
        in_specs=[pl.BlockSpec(memory_space=pltpu.VMEM)],
        out_specs=pl.BlockSpec(memory_space=pltpu.VMEM),
        scratch_shapes=[
            pltpu.VMEM((wn, wm), x.dtype),
            pltpu.VMEM((wn, wm), x.dtype),
            pltpu.VMEM((wn, wm), x.dtype),
            pltpu.SemaphoreType.DMA,
            pltpu.SemaphoreType.DMA,
            pltpu.SemaphoreType.DMA,
            pltpu.SemaphoreType.DMA,
        ],
    )(x)


# device time: 33139 ns/iter; 2.0616x vs baseline; 1.0275x over previous
import jax
import jax.numpy as jnp
from jax import lax
from jax.experimental import pallas as pl
from jax.experimental.pallas import tpu as pltpu

N_DEV = 32
C_PER = 4
LOG_TOT = 13


def kernel(x):
    m, n = x.shape
    assert m == 256 and n == 128

    wn, wm = n, m

    def body(x_ref, o_ref, xt_ref, w_ref, ot_ref, sf_send, sf_recv, sb_send, sb_recv):
        pos = lax.axis_index("i")

        def drain(send_sem, recv_sem, count, which):
            dummy = pltpu.make_async_remote_copy(
                src_ref=xt_ref.at[pl.ds(0, C_PER)],
                dst_ref=xt_ref.at[pl.ds(0, C_PER)],
                send_sem=send_sem,
                recv_sem=recv_sem,
                device_id=(pos,),
                device_id_type=pl.DeviceIdType.MESH,
            )
            for _ in range(count):
                if which == "recv":
                    dummy.wait_recv()
                else:
                    dummy.wait_send()

        xt_ref[:, :] = x_ref[:, :].T

        for d in range(N_DEV):
            q = pos ^ d
            copy = pltpu.make_async_remote_copy(
                src_ref=xt_ref.at[pl.ds(C_PER * q, C_PER)],
                dst_ref=w_ref.at[pl.ds(C_PER * pos, C_PER)],
                send_sem=sf_send,
                recv_sem=sf_recv,
                device_id=(q,),
                device_id_type=pl.DeviceIdType.MESH,
            )

            copy.start()

        drain(sf_send, sf_recv, N_DEV, "recv")

        row = lax.broadcasted_iota(jnp.int32, (wn, wm), 0)
        lane = lax.broadcasted_iota(jnp.int32, (wn, wm), 1)
        fidx = (row // C_PER) * wm + lane

        w = w_ref[:, :]
        for lk in range(1, LOG_TOT + 1):
            k = 1 << lk
            asc = (fidx & k) == 0
            for lj in range(lk - 1, -1, -1):
                j = 1 << lj
                if j >= wm:
                    d = C_PER * (j // wm)
                    up = jnp.roll(w, -d, axis=0)
                    down = jnp.roll(w, d, axis=0)
                else:
                    up = jnp.roll(w, -j, axis=1)
                    down = jnp.roll(w, j, axis=1)
                lower = (fidx & j) == 0
                partner = jnp.where(lower, up, down)
                w = jnp.where(
                    asc == lower,
                    jnp.minimum(w, partner),
                    jnp.maximum(w, partner),
                )
        w_ref[:, :] = w

        for d in range(N_DEV):
            p = pos ^ d
            copy = pltpu.make_async_remote_copy(
                src_ref=w_ref.at[pl.ds(C_PER * p, C_PER)],
                dst_ref=ot_ref.at[pl.ds(C_PER * pos, C_PER)],
                send_sem=sb_send,
                recv_sem=sb_recv,
                device_id=(p,),
                device_id_type=pl.DeviceIdType.MESH,
            )

            copy.start()

        drain(sb_send, sb_recv, N_DEV, "recv")
        o_ref[:, :] = ot_ref[:, :].T

        drain(sf_send, sf_recv, N_DEV, "send")
        drain(sb_send, sb_recv, N_DEV, "send")

    return pl.pallas_call(
        body,
        out_shape=jax.ShapeDtypeStruct((m, n), x.dtype),
        in_specs=[pl.BlockSpec(memory_space=pltpu.VMEM)],
        out_specs=pl.BlockSpec(memory_space=pltpu.VMEM),
        scratch_shapes=[
            pltpu.VMEM((wn, wm), x.dtype),
            pltpu.VMEM((wn, wm), x.dtype),
            pltpu.VMEM((wn, wm), x.dtype),
            pltpu.SemaphoreType.DMA,
            pltpu.SemaphoreType.DMA,
            pltpu.SemaphoreType.DMA,
            pltpu.SemaphoreType.DMA,
        ],
    )(x)


# device time: 24698 ns/iter; 2.7663x vs baseline; 1.3418x over previous
import jax
import jax.numpy as jnp
from jax import lax
from jax.experimental import pallas as pl
from jax.experimental.pallas import tpu as pltpu

N_DEV = 32
C_PER = 4
LOG_TOT = 13


def kernel(x):
    m, n = x.shape
    assert m == 256 and n == 128

    wn, wm = n, m

    def body(x_ref, o_ref, xt_ref, w_ref, ot_ref, sf_send, sf_recv, sb_send, sb_recv):
        pos = lax.axis_index("i")

        def drain(send_sem, recv_sem, count, which):
            dummy = pltpu.make_async_remote_copy(
                src_ref=xt_ref.at[pl.ds(0, C_PER)],
                dst_ref=xt_ref.at[pl.ds(0, C_PER)],
                send_sem=send_sem,
                recv_sem=recv_sem,
                device_id=(pos,),
                device_id_type=pl.DeviceIdType.MESH,
            )
            for _ in range(count):
                if which == "recv":
                    dummy.wait_recv()
                else:
                    dummy.wait_send()

        barrier_sem = pltpu.get_barrier_semaphore()
        for d in range(1, N_DEV):
            pl.semaphore_signal(
                barrier_sem,
                inc=1,
                device_id=(pos ^ d,),
                device_id_type=pl.DeviceIdType.MESH,
            )

        xt_ref[:, :] = x_ref[:, :].T

        pl.semaphore_wait(barrier_sem, N_DEV - 1)

        for d in range(N_DEV):
            q = pos ^ d
            copy = pltpu.make_async_remote_copy(
                src_ref=xt_ref.at[pl.ds(C_PER * q, C_PER)],
                dst_ref=w_ref.at[pl.ds(C_PER * pos, C_PER)],
                send_sem=sf_send,
                recv_sem=sf_recv,
                device_id=(q,),
                device_id_type=pl.DeviceIdType.MESH,
            )

            copy.start()

        drain(sf_send, sf_recv, N_DEV, "recv")

        row = lax.broadcasted_iota(jnp.int32, (wn, wm), 0)
        lane = lax.broadcasted_iota(jnp.int32, (wn, wm), 1)
        fidx = (row // C_PER) * wm + lane

        w = w_ref[:, :]
        for lk in range(1, LOG_TOT + 1):
            k = 1 << lk
            asc = (fidx & k) == 0
            for lj in range(lk - 1, -1, -1):
                j = 1 << lj
                if j >= wm:
                    d = C_PER * (j // wm)
                    up = jnp.roll(w, -d, axis=0)
                    down = jnp.roll(w, d, axis=0)
                else:
                    up = jnp.roll(w, -j, axis=1)
                    down = jnp.roll(w, j, axis=1)
                lower = (fidx & j) == 0
                partner = jnp.where(lower, up, down)
                w = jnp.where(
                    asc == lower,
                    jnp.minimum(w, partner),
                    jnp.maximum(w, partner),
                )
        w_ref[:, :] = w

        for d in range(N_DEV):
            p = pos ^ d
            copy = pltpu.make_async_remote_copy(
                src_ref=w_ref.at[pl.ds(C_PER * p, C_PER)],
                dst_ref=ot_ref.at[pl.ds(C_PER * pos, C_PER)],
                send_sem=sb_send,
                recv_sem=sb_recv,
                device_id=(p,),
                device_id_type=pl.DeviceIdType.MESH,
            )

            copy.start()

        drain(sb_send, sb_recv, N_DEV, "recv")
        o_ref[:, :] = ot_ref[:, :].T

        drain(sf_send, sf_recv, N_DEV, "send")
        drain(sb_send, sb_recv, N_DEV, "send")

    return pl.pallas_call(
        body,
        out_shape=jax.ShapeDtypeStruct((m, n), x.dtype),
        in_specs=[pl.BlockSpec(memory_space=pltpu.VMEM)],
        out_specs=pl.BlockSpec(memory_space=pltpu.VMEM),
        scratch_shapes=[
            pltpu.VMEM((wn, wm), x.dtype),
            pltpu.VMEM((wn, wm), x.dtype),
            pltpu.VMEM((wn, wm), x.dtype),
            pltpu.SemaphoreType.DMA,
            pltpu.SemaphoreType.DMA,
            pltpu.SemaphoreType.DMA,
            pltpu.SemaphoreType.DMA,
        ],
        compiler_params=pltpu.CompilerParams(collective_id=0),
    )(x)
